# baseline (speedup 1.0000x reference)
import functools

import jax
import jax.numpy as jnp
from jax import lax
from jax.experimental import pallas as pl
from jax.experimental.pallas import tpu as pltpu
from jax.experimental.pallas import tpu_sc as plsc

_EMBED_DIMS = [(33, 17), (2, 1), (8, 4), (3, 2), (3, 2), (3, 2), (3, 2), (3, 2), (20, 10)]
_VOFFS = []
_COFFS = []
_v = 0
_c = 0
for _vv, _dd in _EMBED_DIMS:
    _VOFFS.append(_v)
    _COFFS.append(_c)
    _v += _vv
    _c += _dd
_TOTV = _v
_TOTC = _c
_VPAD = 80
_CPAD = 48
_NT = len(_EMBED_DIMS)

_NC, _NS = 2, 16
_NW = _NC * _NS


def _pack_body(nb, e0, e1, e2, e3, e4, e5, e6, e7, e8, w_ref, b_ref, xr_ref,
               g_ref, be_ref, tp_ref, wf_ref, a_ref):
    xr = xr_ref[...]
    mean = jnp.sum(xr) * (1.0 / nb)
    var = jnp.sum((xr - mean) ** 2) * (1.0 / nb)
    a = g_ref[0, 0] * jax.lax.rsqrt(var + 1e-5)
    c = be_ref[0, 0] - mean * a

    tp_ref[...] = jnp.zeros_like(tp_ref)
    for i, t in enumerate([e0, e1, e2, e3, e4, e5, e6, e7, e8]):
        v, d = _EMBED_DIMS[i]
        tp_ref[_VOFFS[i]:_VOFFS[i] + v, _COFFS[i]:_COFFS[i] + d] = t[...]

    wf_ref[...] = jnp.zeros_like(wf_ref)
    wf_ref[:, :_TOTC + 1] = w_ref[...]
    wf_ref[:, _TOTC + 1:_TOTC + 2] = b_ref[...] + c * w_ref[:, _TOTC:_TOTC + 1]
    a_ref[...] = jnp.full((1, 1), a, jnp.float32)


def _sc_body(bpw, nb, tpad_hbm, cat_hbm, out_hbm, catv, tflat, accT, sem_in, sem_out):
    wid = lax.axis_index("s") * _NC + lax.axis_index("c")
    base = wid * bpw
    loads = [pltpu.async_copy(tpad_hbm, tflat, sem_in)]
    for i in range(_NT):
        loads.append(pltpu.async_copy(cat_hbm.at[pl.ds(i * nb + base, bpw)],
                                      catv.at[pl.ds(i * bpw, bpw)], sem_in))
    for cp in loads:
        cp.wait()

    zeros16 = jnp.zeros((16,), jnp.float32)
    ones16 = jnp.ones((16,), jnp.float32)
    for j in range(_TOTC, _CPAD):
        fill = ones16 if j == _TOTC + 1 else zeros16
        for g in range(bpw // 16):
            accT[pl.ds(j * bpw + g * 16, 16)] = fill

    @plsc.parallel_loop(0, bpw // 16, unroll=4)
    def _loop(g):
        for i in range(_NT):
            cv = catv[pl.ds(i * bpw + g * 16, 16)]
            fi = cv * _CPAD + (_VOFFS[i] * _CPAD + _COFFS[i])
            for r in range(_EMBED_DIMS[i][1]):
                vals = plsc.load_gather(tflat, [fi + r])
                accT[pl.ds((_COFFS[i] + r) * bpw + g * 16, 16)] = vals

    stores = [pltpu.async_copy(accT.at[pl.ds(j * bpw, bpw)],
                               out_hbm.at[pl.ds(j * nb + base, bpw)], sem_out)
              for j in range(_CPAD)]
    for cp in stores:
        cp.wait()


def _tc_body(et_ref, xrow_ref, w_ref, a_ref, o_ref):
    cnT = xrow_ref[...] * a_ref[0, 0]
    ih = lax.broadcasted_iota(jnp.int32, et_ref.shape, 0)
    eh = jnp.where(ih == _TOTC, cnT, et_ref[...])
    z = jax.lax.dot_general(
        eh, w_ref[...], (((0,), (1,)), ((), ())),
        preferred_element_type=jnp.float32, precision=jax.lax.Precision.HIGHEST)
    z = z - jnp.max(z, axis=1, keepdims=True)
    ez = jnp.exp(z)
    o_ref[...] = ez / jnp.sum(ez, axis=1, keepdims=True)


def kernel(x, emb0, emb1, emb2, emb3, emb4, emb5, emb6, emb7, emb8, W, b, gamma, beta):
    tables = [emb0, emb1, emb2, emb3, emb4, emb5, emb6, emb7, emb8]
    B = x.shape[0]
    d_out = W.shape[0]
    bpw = B // _NW

    cat_t = x[:, 1:].astype(jnp.int32).T
    xc = x[:, 0]
    xr = xc.reshape(128, B // 128)
    xrow = xc.reshape(1, B)
    b1 = b.reshape(d_out, 1)
    g2 = gamma.reshape(1, 1)
    be2 = beta.reshape(1, 1)

    full = lambda s: pl.BlockSpec(s, lambda: (0,) * len(s))
    tpad, wfull, aa = pl.pallas_call(
        functools.partial(_pack_body, float(B)),
        in_specs=[full(t.shape) for t in tables]
        + [full(W.shape), full(b1.shape), full(xr.shape), full(g2.shape), full(be2.shape)],
        out_specs=[full((_VPAD, _CPAD)), full((d_out, _CPAD)), full((1, 1))],
        out_shape=[
            jax.ShapeDtypeStruct((_VPAD, _CPAD), jnp.float32),
            jax.ShapeDtypeStruct((d_out, _CPAD), jnp.float32),
            jax.ShapeDtypeStruct((1, 1), jnp.float32),
        ],
    )(*tables, W, b1, xr, g2, be2)

    mesh = plsc.VectorSubcoreMesh(core_axis_name="c", subcore_axis_name="s")
    ef = pl.kernel(
        functools.partial(_sc_body, bpw, B),
        out_type=jax.ShapeDtypeStruct((_CPAD * B,), jnp.float32),
        mesh=mesh,
        scratch_types=[
            pltpu.VMEM((_NT * bpw,), jnp.int32),
            pltpu.VMEM((_VPAD * _CPAD,), jnp.float32),
            pltpu.VMEM((bpw * _CPAD,), jnp.float32),
            pltpu.SemaphoreType.DMA,
            pltpu.SemaphoreType.DMA,
        ],
        compiler_params=pltpu.CompilerParams(needs_layout_passes=False),
    )(tpad.reshape(-1), cat_t.reshape(-1))
    et = ef.reshape(_CPAD, B)

    bb = 1024
    out = pl.pallas_call(
        _tc_body,
        grid=(B // bb,),
        in_specs=[
            pl.BlockSpec((_CPAD, bb), lambda i: (0, i)),
            pl.BlockSpec((1, bb), lambda i: (0, i)),
            pl.BlockSpec((d_out, _CPAD), lambda i: (0, 0)),
            pl.BlockSpec((1, 1), lambda i: (0, 0)),
        ],
        out_specs=pl.BlockSpec((bb, d_out), lambda i: (i, 0)),
        out_shape=jax.ShapeDtypeStruct((B, d_out), jnp.float32),
        compiler_params=pltpu.CompilerParams(fuse_transposed_lhs_in_matmul=True),
    )(et, xrow, wfull, aa)
    return out
